# Initial kernel scaffold; baseline (speedup 1.0000x reference)
import jax
import jax.numpy as jnp
from jax.experimental import pallas as pl


def kernel(inputs, embeddings):
    raise NotImplementedError("write your pallas kernel here")


# speedup vs baseline: 18.8300x; 18.8300x over previous
import functools

import jax
import jax.numpy as jnp
from jax import lax
from jax.experimental import pallas as pl
from jax.experimental.pallas import tpu as pltpu
from jax.experimental.pallas import tpu_sc as plsc

_INPUT_DIM = 100000
_D = 32
_BATCH = 4096
_SEQ = 200
_N = _BATCH * _SEQ


def _make_kernel():
    info = plsc.get_sparse_core_info()
    nc, ns = info.num_cores, info.num_subcores
    nw = nc * ns
    n_per_w = _N // nw
    g = 128
    ng = n_per_w // g

    mesh = plsc.VectorSubcoreMesh(core_axis_name="c", subcore_axis_name="s")

    @functools.partial(
        pl.kernel,
        mesh=mesh,
        out_type=jax.ShapeDtypeStruct((_N, _D), jnp.float32),
        scratch_types=[
            pltpu.VMEM((n_per_w,), jnp.int32),
            pltpu.VMEM((g, _D), jnp.float32),
            pltpu.SemaphoreType.DMA,
        ],
        compiler_params=pltpu.CompilerParams(use_tc_tiling_on_sc=False),
    )
    def k(idx_hbm, table_hbm, out_hbm, idx_v, rows_v, sem):
        wid = lax.axis_index("s") * nc + lax.axis_index("c")
        base = wid * n_per_w
        pltpu.sync_copy(idx_hbm.at[pl.ds(base, n_per_w)], idx_v)

        def off_body(i, carry):
            sl = pl.ds(i * 16, 16)
            v = idx_v[sl]
            v = jnp.minimum(jnp.maximum(v, -_INPUT_DIM), _INPUT_DIM) + _INPUT_DIM
            idx_v[sl] = v
            return carry

        lax.fori_loop(0, n_per_w // 16, off_body, 0)

        def g_body(i, carry):
            pltpu.async_copy(
                table_hbm.at[idx_v.at[pl.ds(i * g, g)]], rows_v, sem
            ).wait()
            pltpu.sync_copy(rows_v, out_hbm.at[pl.ds(base + i * g, g)])
            return carry

        lax.fori_loop(0, ng, g_body, 0)

    return k


_gather_kernel = _make_kernel()


def kernel(inputs, embeddings):
    flat = inputs.reshape(_N)
    out = _gather_kernel(flat, embeddings)
    return out.reshape(_BATCH, _SEQ, _D)


# speedup vs baseline: 22.0627x; 1.1717x over previous
import functools

import jax
import jax.numpy as jnp
from jax import lax
from jax.experimental import pallas as pl
from jax.experimental.pallas import tpu as pltpu
from jax.experimental.pallas import tpu_sc as plsc

_INPUT_DIM = 100000
_D = 32
_BATCH = 4096
_SEQ = 200
_N = _BATCH * _SEQ


def _make_kernel():
    info = plsc.get_sparse_core_info()
    nc, ns = info.num_cores, info.num_subcores
    nw = nc * ns
    n_per_w = _N // nw
    g = 512
    ng = n_per_w // g

    mesh = plsc.VectorSubcoreMesh(core_axis_name="c", subcore_axis_name="s")

    @functools.partial(
        pl.kernel,
        mesh=mesh,
        out_type=jax.ShapeDtypeStruct((_N, _D), jnp.float32),
        scratch_types=[
            pltpu.VMEM((n_per_w,), jnp.int32),
            pltpu.VMEM((g, _D), jnp.float32),
            pltpu.SemaphoreType.DMA,
        ],
        compiler_params=pltpu.CompilerParams(use_tc_tiling_on_sc=False),
    )
    def k(idx_hbm, table_hbm, out_hbm, idx_v, rows_v, sem):
        wid = lax.axis_index("s") * nc + lax.axis_index("c")
        base = wid * n_per_w
        pltpu.sync_copy(idx_hbm.at[pl.ds(base, n_per_w)], idx_v)

        def off_body(i, carry):
            sl = pl.ds(i * 16, 16)
            v = idx_v[sl]
            v = jnp.minimum(jnp.maximum(v, -_INPUT_DIM), _INPUT_DIM) + _INPUT_DIM
            idx_v[sl] = v
            return carry

        lax.fori_loop(0, n_per_w // 16, off_body, 0)

        def g_body(i, carry):
            pltpu.async_copy(
                table_hbm.at[idx_v.at[pl.ds(i * g, g)]], rows_v, sem
            ).wait()
            pltpu.sync_copy(rows_v, out_hbm.at[pl.ds(base + i * g, g)])
            return carry

        lax.fori_loop(0, ng, g_body, 0)

    return k


_gather_kernel = _make_kernel()


def kernel(inputs, embeddings):
    flat = inputs.reshape(_N)
    out = _gather_kernel(flat, embeddings)
    return out.reshape(_BATCH, _SEQ, _D)


# speedup vs baseline: 23.3193x; 1.0570x over previous
import functools

import jax
import jax.numpy as jnp
from jax import lax
from jax.experimental import pallas as pl
from jax.experimental.pallas import tpu as pltpu
from jax.experimental.pallas import tpu_sc as plsc

_INPUT_DIM = 100000
_D = 32
_BATCH = 4096
_SEQ = 200
_N = _BATCH * _SEQ


def _make_kernel():
    info = plsc.get_sparse_core_info()
    nc, ns = info.num_cores, info.num_subcores
    nw = nc * ns
    n_per_w = _N // nw
    g = 800
    ng = n_per_w // g
    nbuf = 4

    mesh = plsc.VectorSubcoreMesh(core_axis_name="c", subcore_axis_name="s")

    @functools.partial(
        pl.kernel,
        mesh=mesh,
        out_type=jax.ShapeDtypeStruct((_N, _D), jnp.float32),
        scratch_types=[
            pltpu.VMEM((n_per_w,), jnp.int32),
            pltpu.VMEM((nbuf, g, _D), jnp.float32),
            pltpu.SemaphoreType.DMA,
            pltpu.SemaphoreType.DMA,
            pltpu.SemaphoreType.DMA,
            pltpu.SemaphoreType.DMA,
            pltpu.SemaphoreType.DMA,
            pltpu.SemaphoreType.DMA,
            pltpu.SemaphoreType.DMA,
            pltpu.SemaphoreType.DMA,
        ],
        compiler_params=pltpu.CompilerParams(use_tc_tiling_on_sc=False),
    )
    def k(idx_hbm, table_hbm, out_hbm, idx_v, rows_v,
          g0, g1, g2, g3, o0, o1, o2, o3):
        gsem = (g0, g1, g2, g3)
        osem = (o0, o1, o2, o3)
        wid = lax.axis_index("s") * nc + lax.axis_index("c")
        base = wid * n_per_w
        pltpu.sync_copy(idx_hbm.at[pl.ds(base, n_per_w)], idx_v)

        def off_body(t, carry):
            sl = pl.ds(t * 16, 16)
            v = idx_v[sl]
            idx_v[sl] = (
                jnp.minimum(jnp.maximum(v, -_INPUT_DIM), _INPUT_DIM) + _INPUT_DIM
            )
            return carry

        lax.fori_loop(0, n_per_w // 16, off_body, 0)

        def gather_desc(c, b):
            return pltpu.make_async_copy(
                table_hbm.at[idx_v.at[pl.ds(c * g, g)]], rows_v.at[b], gsem[b]
            )

        def out_desc(c, b):
            return pltpu.make_async_copy(
                rows_v.at[b], out_hbm.at[pl.ds(base + c * g, g)], osem[b]
            )

        def step(i, b, bc, wait_prev_out, issue_next):
            gather_desc(i, b).wait()
            out_desc(i, b).start()
            if issue_next:
                c = i + nbuf - 1
                if wait_prev_out:
                    out_desc(c - nbuf, bc).wait()
                gather_desc(c, bc).start()

        for b in range(nbuf - 1):
            gather_desc(b, b).start()
        step(0, 0, 3, False, True)
        step(1, 1, 0, True, True)
        step(2, 2, 1, True, True)
        step(3, 3, 2, True, True)

        def loop_body(j, carry):
            i0 = j * nbuf
            step(i0 + 0, 0, 3, True, True)
            step(i0 + 1, 1, 0, True, True)
            step(i0 + 2, 2, 1, True, True)
            step(i0 + 3, 3, 2, True, True)
            return carry

        lax.fori_loop(1, ng // nbuf - 1, loop_body, 0)

        step(ng - 4, 0, 3, True, True)
        step(ng - 3, 1, 0, False, False)
        step(ng - 2, 2, 1, False, False)
        step(ng - 1, 3, 2, False, False)
        out_desc(ng - 4, 0).wait()
        out_desc(ng - 3, 1).wait()
        out_desc(ng - 2, 2).wait()
        out_desc(ng - 1, 3).wait()

    return k


_gather_kernel = _make_kernel()


def kernel(inputs, embeddings):
    flat = inputs.reshape(_N)
    out = _gather_kernel(flat, embeddings)
    return out.reshape(_BATCH, _SEQ, _D)


# speedup vs baseline: 42.6395x; 1.8285x over previous
import functools

import jax
import jax.numpy as jnp
from jax import lax
from jax.experimental import pallas as pl
from jax.experimental.pallas import tpu as pltpu
from jax.experimental.pallas import tpu_sc as plsc

_INPUT_DIM = 100000
_D = 32
_BATCH = 4096
_SEQ = 200
_N = _BATCH * _SEQ


def _make_kernel():
    info = plsc.get_sparse_core_info()
    nc, ns = info.num_cores, info.num_subcores
    nw = nc * ns
    n_per_w = _N // nw
    gs = 4
    rows = gs * 128
    ng = _SEQ // gs

    mesh = plsc.VectorSubcoreMesh(core_axis_name="c", subcore_axis_name="s")

    @functools.partial(
        pl.kernel,
        mesh=mesh,
        out_type=jax.ShapeDtypeStruct((_SEQ, 4, 32, 8, 128), jnp.float32),
        scratch_types=[
            pltpu.VMEM((n_per_w,), jnp.int32),
            pltpu.VMEM((n_per_w,), jnp.int32),
            pltpu.VMEM((2, rows, _D), jnp.float32),
            pltpu.VMEM((2, gs, 4, 8, 129), jnp.float32),
            pltpu.SemaphoreType.DMA,
            pltpu.SemaphoreType.DMA,
            pltpu.SemaphoreType.DMA,
            pltpu.SemaphoreType.DMA,
        ],
        compiler_params=pltpu.CompilerParams(
            use_tc_tiling_on_sc=False, needs_layout_passes=False
        ),
    )
    def k(idx_hbm, table_hbm, out_hbm, idx_v, gidx_v, gbuf, tbuf,
          g0, g1, o0, o1):
        gsem = (g0, g1)
        osem = (o0, o1)
        w = lax.axis_index("s") * nc + lax.axis_index("c")
        base = w * n_per_w
        pltpu.sync_copy(idx_hbm.at[pl.ds(base, n_per_w)], idx_v)

        iota = lax.iota(jnp.int32, 16)
        iota200 = iota * _SEQ
        zeros = jnp.zeros((16,), jnp.int32)
        dgv = (iota >> 3, (iota >> 3) + 2)
        ddv = iota & 7

        def build_body(s, carry):
            for q in range(8):
                inds = iota200 + (q * 16 * _SEQ + s)
                v = plsc.load_gather(idx_v, [inds])
                v = (
                    jnp.minimum(jnp.maximum(v, -_INPUT_DIM), _INPUT_DIM)
                    + _INPUT_DIM
                )
                gidx_v[pl.ds(s * 128 + q * 16, 16)] = v
            return carry

        lax.fori_loop(0, _SEQ, build_body, 0)

        def gather_desc(i, p):
            return pltpu.make_async_copy(
                table_hbm.at[gidx_v.at[pl.ds(i * rows, rows)]],
                gbuf.at[p],
                gsem[p],
            )

        def out_desc(i, p):
            return pltpu.make_async_copy(
                tbuf.at[p, :, :, :, pl.ds(0, 128)],
                out_hbm.at[pl.ds(i * gs, gs), :, w],
                osem[p],
            )

        def transpose_chunk(p):
            def u_body(u, carry):
                sl = u >> 3
                q = u & 7
                r0 = u * 16
                sl_s = zeros + sl
                for kk in range(16):
                    r = r0 + kk
                    bl_s = zeros + (q * 16 + kk)
                    for h in range(2):
                        v = gbuf[p, r, pl.ds(16 * h, 16)]
                        plsc.store_scatter(
                            tbuf.at[p], [sl_s, dgv[h], ddv, bl_s], v
                        )
                return carry

            lax.fori_loop(0, rows // 16, u_body, 0)

        def step(i, p, wait_prev, issue_next):
            gather_desc(i, p).wait()
            if wait_prev:
                out_desc(i - 2, p).wait()
            transpose_chunk(p)
            out_desc(i, p).start()
            if issue_next:
                gather_desc(i + 2, p).start()

        gather_desc(0, 0).start()
        gather_desc(1, 1).start()
        step(0, 0, False, True)
        step(1, 1, False, True)

        def loop_body(j, carry):
            step(2 * j, 0, True, True)
            step(2 * j + 1, 1, True, True)
            return carry

        lax.fori_loop(1, ng // 2 - 1, loop_body, 0)

        step(ng - 2, 0, True, False)
        step(ng - 1, 1, True, False)
        out_desc(ng - 2, 0).wait()
        out_desc(ng - 1, 1).wait()

    return k


_gather_kernel = _make_kernel()


def kernel(inputs, embeddings):
    flat = inputs.reshape(_N)
    out5 = _gather_kernel(flat, embeddings)
    return out5.transpose(2, 4, 0, 1, 3).reshape(_BATCH, _SEQ, _D)


# speedup vs baseline: 54.6332x; 1.2813x over previous
import functools

import jax
import jax.numpy as jnp
from jax import lax
from jax.experimental import pallas as pl
from jax.experimental.pallas import tpu as pltpu
from jax.experimental.pallas import tpu_sc as plsc

_INPUT_DIM = 100000
_D = 32
_BATCH = 4096
_SEQ = 200
_N = _BATCH * _SEQ


def _make_kernel():
    info = plsc.get_sparse_core_info()
    nc, ns = info.num_cores, info.num_subcores
    nw = nc * ns
    n_per_w = _N // nw
    gs = 4
    rows = gs * 128
    ng = _SEQ // gs

    mesh = plsc.VectorSubcoreMesh(core_axis_name="c", subcore_axis_name="s")

    @functools.partial(
        pl.kernel,
        mesh=mesh,
        out_type=jax.ShapeDtypeStruct((_SEQ, 4, 32, 8, 128), jnp.float32),
        scratch_types=[
            pltpu.VMEM((n_per_w,), jnp.int32),
            pltpu.VMEM((n_per_w,), jnp.int32),
            pltpu.VMEM((2, rows, _D), jnp.float32),
            pltpu.VMEM((2, gs, 4, 8, 129), jnp.float32),
            pltpu.SemaphoreType.DMA,
            pltpu.SemaphoreType.DMA,
            pltpu.SemaphoreType.DMA,
            pltpu.SemaphoreType.DMA,
        ],
        compiler_params=pltpu.CompilerParams(
            use_tc_tiling_on_sc=False, needs_layout_passes=False
        ),
    )
    def k(idx_hbm, table_hbm, out_hbm, idx_v, gidx_v, gbuf, tbuf,
          g0, g1, o0, o1):
        gsem = (g0, g1)
        osem = (o0, o1)
        w = lax.axis_index("s") * nc + lax.axis_index("c")
        base = w * n_per_w
        pltpu.sync_copy(idx_hbm.at[pl.ds(base, n_per_w)], idx_v)

        iota = lax.iota(jnp.int32, 16)
        iota200 = iota * _SEQ
        zeros = jnp.zeros((16,), jnp.int32)
        iota129 = iota * 129

        def build_body(s, carry):
            for q in range(8):
                inds = iota200 + (q * 16 * _SEQ + s)
                v = plsc.load_gather(idx_v, [inds])
                v = (
                    jnp.minimum(jnp.maximum(v, -_INPUT_DIM), _INPUT_DIM)
                    + _INPUT_DIM
                )
                gidx_v[pl.ds(s * 128 + q * 16, 16)] = v
            return carry

        lax.fori_loop(0, _SEQ, build_body, 0)

        def gather_desc(i, p):
            return pltpu.make_async_copy(
                table_hbm.at[gidx_v.at[pl.ds(i * rows, rows)]],
                gbuf.at[p],
                gsem[p],
            )

        def out_desc(i, p):
            return pltpu.make_async_copy(
                tbuf.at[p, :, :, :, pl.ds(0, 128)],
                out_hbm.at[pl.ds(i * gs, gs), :, w],
                osem[p],
            )

        def transpose_chunk(p):
            def u_body(u, carry):
                sl = u >> 3
                q = u & 7
                r0 = u * 16
                b0 = sl * 4128 + q * 16
                for kk in range(16):
                    r = r0 + kk
                    i0 = iota129 + (b0 + kk)
                    i1 = i0 + 2064
                    v0 = gbuf[p, r, pl.ds(0, 16)]
                    v1 = gbuf[p, r, pl.ds(16, 16)]
                    plsc.store_scatter(tbuf.at[p], [zeros, zeros, zeros, i0], v0)
                    plsc.store_scatter(tbuf.at[p], [zeros, zeros, zeros, i1], v1)
                return carry

            lax.fori_loop(0, rows // 16, u_body, 0)

        def step(i, p, wait_prev, issue_next):
            gather_desc(i, p).wait()
            if wait_prev:
                out_desc(i - 2, p).wait()
            transpose_chunk(p)
            out_desc(i, p).start()
            if issue_next:
                gather_desc(i + 2, p).start()

        gather_desc(0, 0).start()
        gather_desc(1, 1).start()
        step(0, 0, False, True)
        step(1, 1, False, True)

        def loop_body(j, carry):
            step(2 * j, 0, True, True)
            step(2 * j + 1, 1, True, True)
            return carry

        lax.fori_loop(1, ng // 2 - 1, loop_body, 0)

        step(ng - 2, 0, True, False)
        step(ng - 1, 1, True, False)
        out_desc(ng - 2, 0).wait()
        out_desc(ng - 1, 1).wait()

    return k


_gather_kernel = _make_kernel()


def kernel(inputs, embeddings):
    flat = inputs.reshape(_N)
    out5 = _gather_kernel(flat, embeddings)
    return out5.transpose(2, 4, 0, 1, 3).reshape(_BATCH, _SEQ, _D)


# speedup vs baseline: 58.7032x; 1.0745x over previous
import functools

import jax
import jax.numpy as jnp
from jax import lax
from jax.experimental import pallas as pl
from jax.experimental.pallas import tpu as pltpu
from jax.experimental.pallas import tpu_sc as plsc

_INPUT_DIM = 100000
_D = 32
_BATCH = 4096
_SEQ = 200
_N = _BATCH * _SEQ


def _make_kernel():
    info = plsc.get_sparse_core_info()
    nc, ns = info.num_cores, info.num_subcores
    nw = nc * ns
    n_per_w = _N // nw
    gs = 4
    rows = gs * 128
    ng = _SEQ // gs

    mesh = plsc.VectorSubcoreMesh(core_axis_name="c", subcore_axis_name="s")

    @functools.partial(
        pl.kernel,
        mesh=mesh,
        out_type=jax.ShapeDtypeStruct((_SEQ, 4, 32, 1024), jnp.float32),
        scratch_types=[
            pltpu.VMEM((n_per_w,), jnp.int32),
            pltpu.VMEM((n_per_w,), jnp.int32),
            pltpu.VMEM((2, rows, _D), jnp.float32),
            pltpu.VMEM((2, gs, 4, 1032), jnp.float32),
            pltpu.SemaphoreType.DMA,
            pltpu.SemaphoreType.DMA,
            pltpu.SemaphoreType.DMA,
            pltpu.SemaphoreType.DMA,
        ],
        compiler_params=pltpu.CompilerParams(
            use_tc_tiling_on_sc=False, needs_layout_passes=False
        ),
    )
    def k(idx_hbm, table_hbm, out_hbm, idx_v, gidx_v, gbuf, tbuf,
          g0, g1, o0, o1):
        gsem = (g0, g1)
        osem = (o0, o1)
        w = lax.axis_index("s") * nc + lax.axis_index("c")
        base = w * n_per_w
        pltpu.sync_copy(idx_hbm.at[pl.ds(base, n_per_w)], idx_v)

        iota = lax.iota(jnp.int32, 16)
        iota200 = iota * _SEQ
        zeros = jnp.zeros((16,), jnp.int32)
        dgv = (iota >> 3, (iota >> 3) + 2)
        dd129 = (iota & 7) * 129

        def build_body(s, carry):
            for q in range(8):
                inds = iota200 + (q * 16 * _SEQ + s)
                v = plsc.load_gather(idx_v, [inds])
                v = (
                    jnp.minimum(jnp.maximum(v, -_INPUT_DIM), _INPUT_DIM)
                    + _INPUT_DIM
                )
                gidx_v[pl.ds(s * 128 + q * 16, 16)] = v
            return carry

        lax.fori_loop(0, _SEQ, build_body, 0)

        def gather_desc(i, p):
            return pltpu.make_async_copy(
                table_hbm.at[gidx_v.at[pl.ds(i * rows, rows)]],
                gbuf.at[p],
                gsem[p],
            )

        def out_desc(i, p):
            return pltpu.make_async_copy(
                tbuf.at[p, :, :, pl.ds(0, 1024)],
                out_hbm.at[pl.ds(i * gs, gs), :, w],
                osem[p],
            )

        def transpose_chunk(p):
            def u_body(u, carry):
                sl = u >> 3
                q = u & 7
                r0 = u * 16
                sl_s = zeros + sl
                b0 = q * 16
                for kk in range(16):
                    r = r0 + kk
                    mn = dd129 + (b0 + kk)
                    v0 = gbuf[p, r, pl.ds(0, 16)]
                    v1 = gbuf[p, r, pl.ds(16, 16)]
                    plsc.store_scatter(tbuf.at[p], [sl_s, dgv[0], mn], v0)
                    plsc.store_scatter(tbuf.at[p], [sl_s, dgv[1], mn], v1)
                return carry

            lax.fori_loop(0, rows // 16, u_body, 0)

        def step(i, p, wait_prev, issue_next):
            gather_desc(i, p).wait()
            if wait_prev:
                out_desc(i - 2, p).wait()
            transpose_chunk(p)
            out_desc(i, p).start()
            if issue_next:
                gather_desc(i + 2, p).start()

        gather_desc(0, 0).start()
        gather_desc(1, 1).start()
        step(0, 0, False, True)
        step(1, 1, False, True)

        def loop_body(j, carry):
            step(2 * j, 0, True, True)
            step(2 * j + 1, 1, True, True)
            return carry

        lax.fori_loop(1, ng // 2 - 1, loop_body, 0)

        step(ng - 2, 0, True, False)
        step(ng - 1, 1, True, False)
        out_desc(ng - 2, 0).wait()
        out_desc(ng - 1, 1).wait()

    return k


_gather_kernel = _make_kernel()


def kernel(inputs, embeddings):
    flat = inputs.reshape(_N)
    out4 = _gather_kernel(flat, embeddings)
    out5 = out4.reshape(_SEQ, 4, 32, 8, 128)
    return out5.transpose(2, 4, 0, 1, 3).reshape(_BATCH, _SEQ, _D)
